# Initial kernel scaffold; baseline (speedup 1.0000x reference)
import jax
import jax.numpy as jnp
from jax.experimental import pallas as pl


def kernel(x, params):
    raise NotImplementedError("write your pallas kernel here")


# speedup vs baseline: 12.2683x; 12.2683x over previous
import functools

import jax
import jax.numpy as jnp
from jax.experimental import pallas as pl

B = 128
N = 55
D = 3
H = 64
L = 4
NP = 56
DP = 8
CRL = 15.0 / L


def _egnn_kernel(c_ref, h0_ref, w_ref, v_ref, out_ref):
    f32 = jnp.float32
    silu = jax.nn.silu

    coords0 = c_ref[0]
    coords = coords0
    h = jnp.broadcast_to(h0_ref[0:1, :], (NP, H))

    ii = jax.lax.broadcasted_iota(jnp.int32, (NP, NP), 0)
    jj = jax.lax.broadcasted_iota(jnp.int32, (NP, NP), 1)
    emask = ((ii != jj) & (jj < N)).astype(f32)
    emask3 = emask[:, :, None]
    rmask = (jax.lax.broadcasted_iota(jnp.int32, (NP, 1), 0) < N).astype(f32)

    for l in range(L):
        A = w_ref[l, 0]
        Bm = w_ref[l, 1]
        We2 = w_ref[l, 2]
        Wc1 = w_ref[l, 3]
        Wn1a = w_ref[l, 4]
        Wn1b = w_ref[l, 5]
        Wn2 = w_ref[l, 6]
        wr = v_ref[l, 0:1, :]
        be1 = v_ref[l, 1:2, :]
        be2 = v_ref[l, 2:3, :]
        bc1 = v_ref[l, 3:4, :]
        bn1 = v_ref[l, 4:5, :]
        bn2 = v_ref[l, 5:6, :]
        wa = v_ref[l, 6:7, :]
        wc2 = v_ref[l, 7:8, :]
        ba = v_ref[l, 8:9, 0:1]

        diff = coords[:, None, :] - coords[None, :, :]
        radial = jnp.sum(diff * diff, axis=2)

        hA = jnp.dot(h, A, preferred_element_type=f32)
        hB = jnp.dot(h, Bm, preferred_element_type=f32)
        pre = (hA[:, None, :] + hB[None, :, :]
               + radial[:, :, None] * wr.reshape(1, 1, H)
               + be1.reshape(1, 1, H))
        m1 = silu(pre).reshape(NP * NP, H)
        m2 = silu(jnp.dot(m1, We2, preferred_element_type=f32) + be2)
        m2 = m2.reshape(NP, NP, H)
        gate = jax.nn.sigmoid(
            jnp.sum(m2 * wa.reshape(1, 1, H), axis=2) + ba)
        m = m2 * gate[:, :, None]

        t = silu(jnp.dot(m.reshape(NP * NP, H), Wc1,
                         preferred_element_type=f32) + bc1).reshape(NP, NP, H)
        phi = jnp.tanh(jnp.sum(t * wc2.reshape(1, 1, H), axis=2)) * CRL
        phi = phi * emask

        coords = coords + jnp.sum(diff * phi[:, :, None], axis=1)
        m_agg = jnp.sum(m * emask3, axis=1)

        hp = silu(jnp.dot(h, Wn1a, preferred_element_type=f32)
                  + jnp.dot(m_agg, Wn1b, preferred_element_type=f32) + bn1)
        h = h + jnp.dot(hp, Wn2, preferred_element_type=f32) + bn2

    vel = coords - coords0
    mean = jnp.sum(vel * rmask, axis=0, keepdims=True) * (1.0 / N)
    out_ref[0] = vel - mean


@jax.jit
def kernel(x, params):
    coords = x.reshape(B, N, D)
    coords = jnp.pad(coords, ((0, 0), (0, NP - N), (0, DP - D)))

    h0 = (params['emb_w'] + params['emb_b'][None, :]).reshape(1, H)
    h0 = jnp.pad(h0, ((0, 7), (0, 0)))

    Ws, Vs = [], []
    for p in params['layers']:
        We1 = p['We1']
        Wn1 = p['Wn1']
        Ws.append(jnp.stack([
            We1[:H], We1[H:2 * H], p['We2'], p['Wc1'],
            Wn1[:H], Wn1[H:], p['Wn2'],
        ]))
        vec = jnp.stack([
            We1[2 * H], p['be1'], p['be2'], p['bc1'], p['bn1'], p['bn2'],
            p['Wa'][:, 0], p['Wc2'][:, 0],
            jnp.broadcast_to(p['ba'], (H,)),
        ])
        Vs.append(jnp.pad(vec, ((0, 7), (0, 0))))
    Wstk = jnp.stack(Ws)
    Vstk = jnp.stack(Vs)

    grid = (B,)
    out = pl.pallas_call(
        _egnn_kernel,
        grid=grid,
        in_specs=[
            pl.BlockSpec((1, NP, DP), lambda b: (b, 0, 0)),
            pl.BlockSpec((8, H), lambda b: (0, 0)),
            pl.BlockSpec((L, 7, H, H), lambda b: (0, 0, 0, 0)),
            pl.BlockSpec((L, 16, H), lambda b: (0, 0, 0)),
        ],
        out_specs=pl.BlockSpec((1, NP, DP), lambda b: (b, 0, 0)),
        out_shape=jax.ShapeDtypeStruct((B, NP, DP), jnp.float32),
    )(coords, h0, Wstk, Vstk)

    vel = out[:, :N, :D].reshape(B, N * D)
    return vel * params['output_scale']
